# baseline (speedup 1.0000x reference)
import functools

import jax
import jax.numpy as jnp
from jax import lax
from jax.experimental import pallas as pl
from jax.experimental.pallas import tpu as pltpu
from jax.experimental.pallas import tpu_sc as plsc

_N = 10000
_E = 320000
_H = 8
_C = 16
_HC = 128
_G = 64

_NPAD = 10112
_NC = 2
_NS = 16
_NW = _NC * _NS
_EPW = 10240
_BB = 64
_NBLK = _EPW // _BB
_GB = 8
_RPS = _NPAD // _NS

_RB = 128
_NRB = _NPAD // _RB

f32 = jnp.float32
i32 = jnp.int32


def _dense_body(x_ref, w_ref, as_ref, ad_ref, h_ref, s_ref, d_ref):
    h = jnp.dot(x_ref[...], w_ref[...], preferred_element_type=f32)
    h_ref[...] = h
    s_ref[...] = jnp.dot(h, as_ref[...], preferred_element_type=f32)
    d_ref[...] = jnp.dot(h, ad_ref[...], preferred_element_type=f32)


_dense_call = pl.pallas_call(
    _dense_body,
    grid=(_NRB,),
    in_specs=[
        pl.BlockSpec((_RB, _HC), lambda i: (i, 0)),
        pl.BlockSpec((_HC, _HC), lambda i: (0, 0)),
        pl.BlockSpec((_HC, 16), lambda i: (0, 0)),
        pl.BlockSpec((_HC, 16), lambda i: (0, 0)),
    ],
    out_specs=[
        pl.BlockSpec((_RB, _HC), lambda i: (i, 0)),
        pl.BlockSpec((_RB, 16), lambda i: (i, 0)),
        pl.BlockSpec((_RB, 16), lambda i: (i, 0)),
    ],
    out_shape=[
        jax.ShapeDtypeStruct((_NPAD, _HC), f32),
        jax.ShapeDtypeStruct((_NPAD, 16), f32),
        jax.ShapeDtypeStruct((_NPAD, 16), f32),
    ],
)


def _mid_body(pa_ref, pd_ref, bias_ref, exp_ref, w_ref, as_ref, ad_ref,
              h_ref, s_ref, d_ref):
    acc = pa_ref[0] + pa_ref[1]
    den16 = pd_ref[0] + pd_ref[1]
    den = jnp.dot(den16, exp_ref[...], preferred_element_type=f32)
    x_in = acc / (den + 1e-16) + bias_ref[...]
    x_in = jnp.maximum(x_in, 0.0)
    h = jnp.dot(x_in, w_ref[...], preferred_element_type=f32)
    h_ref[...] = h
    s_ref[...] = jnp.dot(h, as_ref[...], preferred_element_type=f32)
    d_ref[...] = jnp.dot(h, ad_ref[...], preferred_element_type=f32)


_mid_call = pl.pallas_call(
    _mid_body,
    grid=(_NRB,),
    in_specs=[
        pl.BlockSpec((_NC, _RB, _HC), lambda i: (0, i, 0)),
        pl.BlockSpec((_NC, _RB, 16), lambda i: (0, i, 0)),
        pl.BlockSpec((1, _HC), lambda i: (0, 0)),
        pl.BlockSpec((16, _HC), lambda i: (0, 0)),
        pl.BlockSpec((_HC, _HC), lambda i: (0, 0)),
        pl.BlockSpec((_HC, 16), lambda i: (0, 0)),
        pl.BlockSpec((_HC, 16), lambda i: (0, 0)),
    ],
    out_specs=[
        pl.BlockSpec((_RB, _HC), lambda i: (i, 0)),
        pl.BlockSpec((_RB, 16), lambda i: (i, 0)),
        pl.BlockSpec((_RB, 16), lambda i: (i, 0)),
    ],
    out_shape=[
        jax.ShapeDtypeStruct((_NPAD, _HC), f32),
        jax.ShapeDtypeStruct((_NPAD, 16), f32),
        jax.ShapeDtypeStruct((_NPAD, 16), f32),
    ],
)


def _pool_body(pa_ref, pd_ref, bias_ref, exp_ref, batch_ref, fcw_ref, fcb_ref,
               out_ref, sums_ref, cnt_ref):
    i = pl.program_id(0)

    @pl.when(i == 0)
    def _():
        sums_ref[...] = jnp.zeros_like(sums_ref)
        cnt_ref[...] = jnp.zeros_like(cnt_ref)

    acc = pa_ref[0] + pa_ref[1]
    den16 = pd_ref[0] + pd_ref[1]
    den = jnp.dot(den16, exp_ref[...], preferred_element_type=f32)
    node = acc / (den + 1e-16) + bias_ref[...]
    b_ids = batch_ref[0]
    gids = lax.broadcasted_iota(i32, (_G, _RB), 0)
    mask = (b_ids == gids).astype(f32)
    sums_ref[...] += jnp.dot(mask, node, preferred_element_type=f32)
    cnt_ref[...] += jnp.sum(mask, axis=1, keepdims=True)

    @pl.when(i == _NRB - 1)
    def _():
        pooled = sums_ref[...] / jnp.maximum(cnt_ref[...], 1.0)
        out_ref[...] = (jnp.sum(pooled * fcw_ref[...], axis=1, keepdims=True)
                        + fcb_ref[0, 0])


_pool_call = pl.pallas_call(
    _pool_body,
    grid=(_NRB,),
    in_specs=[
        pl.BlockSpec((_NC, _RB, _HC), lambda i: (0, i, 0)),
        pl.BlockSpec((_NC, _RB, 16), lambda i: (0, i, 0)),
        pl.BlockSpec((1, _HC), lambda i: (0, 0)),
        pl.BlockSpec((16, _HC), lambda i: (0, 0)),
        pl.BlockSpec((1, 1, _RB), lambda i: (i, 0, 0)),
        pl.BlockSpec((1, _HC), lambda i: (0, 0)),
        pl.BlockSpec((1, 1), lambda i: (0, 0)),
    ],
    out_specs=pl.BlockSpec((_G, 1), lambda i: (0, 0)),
    out_shape=jax.ShapeDtypeStruct((_G, 1), f32),
    scratch_shapes=[
        pltpu.VMEM((_G, _HC), f32),
        pltpu.VMEM((_G, 1), f32),
    ],
)


@functools.lru_cache(maxsize=None)
def _make_edge_pass():
  mesh = plsc.VectorSubcoreMesh(core_axis_name="c", subcore_axis_name="s",
                                num_cores=_NC, num_subcores=_NS)

  ring_buf = [
      pltpu.VMEM((_BB,), i32),
      pltpu.VMEM((_BB,), i32),
      pltpu.VMEM((_BB, 16), f32),
      pltpu.VMEM((_BB, 16), f32),
      pltpu.VMEM((_BB, _HC), f32),
      pltpu.VMEM((_BB, 16), f32),
      pltpu.SemaphoreType.DMA,
      pltpu.SemaphoreType.DMA,
      pltpu.SemaphoreType.DMA,
  ]

  @functools.partial(
    pl.kernel,
    out_type=(jax.ShapeDtypeStruct((_NC, _NPAD, _HC), f32),
              jax.ShapeDtypeStruct((_NC, _NPAD, 16), f32)),
    mesh=mesh,
    compiler_params=pltpu.CompilerParams(use_tc_tiling_on_sc=False),
    scratch_types=[
        pltpu.VMEM((_GB, _BB), i32),
        pltpu.VMEM((_GB, _BB), i32),
        pltpu.VMEM_SHARED((_NPAD, _HC), f32),
        pltpu.VMEM_SHARED((_NPAD, 16), f32),
    ] + ring_buf * 3,
  )
  def _edge_pass(src_h, dst_h, h_h, s_h, d_h, acc_o, den_o,
                 srcg, dstg, accs, dens, *ring):
    bufs = [ring[9 * k:9 * (k + 1)] for k in range(3)]
    cid = lax.axis_index("c")
    sid = lax.axis_index("s")
    wid = sid * _NC + cid

    hr0, wb0 = bufs[0][4], bufs[0][5]
    zv = jnp.zeros((16,), f32)

    def zi(k, _):
        for j in range(_HC // 16):
            hr0[k, pl.ds(16 * j, 16)] = zv
        wb0[k, :] = zv
        return 0

    lax.fori_loop(0, _BB, zi, 0)
    r0 = sid * _RPS
    for t in range(_RPS // _BB):
        pltpu.sync_copy(hr0, accs.at[pl.ds(r0 + t * _BB, _BB)])
        pltpu.sync_copy(wb0, dens.at[pl.ds(r0 + t * _BB, _BB)])
    rem = _RPS - (_RPS // _BB) * _BB
    pltpu.sync_copy(hr0.at[pl.ds(0, rem)],
                    accs.at[pl.ds(r0 + _RPS - rem, rem)])
    pltpu.sync_copy(wb0.at[pl.ds(0, rem)],
                    dens.at[pl.ds(r0 + _RPS - rem, rem)])
    plsc.subcore_barrier()

    base_blk = wid * _NBLK
    nblk = jnp.minimum(_NBLK, jnp.maximum(0, (_E - wid * _EPW) // _BB))

    def load_group(first_blk):
        g = pl.multiple_of(base_blk + first_blk, _GB)
        pltpu.sync_copy(src_h.at[pl.ds(g, _GB)], srcg)
        pltpu.sync_copy(dst_h.at[pl.ds(g, _GB)], dstg)

    def stage(j, k):
        sv, dv = bufs[k][0], bufs[k][1]
        row = lax.rem(j, _GB)
        for m in range(_BB // 16):
            sl = pl.ds(16 * m, 16)
            sv[sl] = srcg[row, sl]
            dv[sl] = dstg[row, sl]
        sr, dr, hr = bufs[k][2], bufs[k][3], bufs[k][4]
        g_sd, g_h = bufs[k][6], bufs[k][7]
        pltpu.async_copy(s_h.at[sv], sr, g_sd)
        pltpu.async_copy(d_h.at[dv], dr, g_sd)
        pltpu.async_copy(h_h.at[sv], hr, g_h)

    def wait_gathers_sd(k):
        sv, dv, sr, dr = bufs[k][0], bufs[k][1], bufs[k][2], bufs[k][3]
        g_sd = bufs[k][6]
        pltpu.make_async_copy(s_h.at[sv], sr, g_sd).wait()
        pltpu.make_async_copy(d_h.at[dv], dr, g_sd).wait()

    def wait_gather_h(k):
        sv, hr, g_h = bufs[k][0], bufs[k][4], bufs[k][7]
        pltpu.make_async_copy(h_h.at[sv], hr, g_h).wait()

    def issue_scatter(k):
        dv, hr, wb, g_sc = bufs[k][1], bufs[k][4], bufs[k][5], bufs[k][8]
        pltpu.async_copy(hr, accs.at[dv], g_sc, add=True)
        pltpu.async_copy(wb, dens.at[dv], g_sc, add=True)

    def wait_scatter(k):
        dv, hr, wb, g_sc = bufs[k][1], bufs[k][4], bufs[k][5], bufs[k][8]
        pltpu.make_async_copy(hr, accs.at[dv], g_sc).wait()
        pltpu.make_async_copy(wb, dens.at[dv], g_sc).wait()

    def compute(k):
        sr, dr, hr, wb = bufs[k][2], bufs[k][3], bufs[k][4], bufs[k][5]

        @plsc.parallel_loop(0, _BB, unroll=4)
        def edge_w(b):
            e = sr[b, :] + dr[b, :]
            e = jnp.where(e > 0, e, f32(0.2) * e)
            wb[b, :] = jnp.exp(e)

        wait_gather_h(k)

        def edge_m(b, _):
            w = wb[b, :]
            for j in range(_H):
                hr[b, pl.ds(16 * j, 16)] = hr[b, pl.ds(16 * j, 16)] * w[j]
            return 0

        lax.fori_loop(0, _BB, edge_m, 0)

    load_group(0)
    stage(0, 0)
    stage(1, 1)

    def step(i3, _):
        for k in range(3):
            j = 3 * i3 + k

            @pl.when(j < nblk)
            def _():
                k2 = (k + 2) % 3

                @pl.when(j + 2 < nblk)
                def _():
                    @pl.when(j >= 1)
                    def _():
                        wait_scatter(k2)

                    @pl.when(lax.rem(j + 2, _GB) == 0)
                    def _():
                        load_group(j + 2)

                    stage(j + 2, k2)

                wait_gathers_sd(k)
                compute(k)
                issue_scatter(k)

        return 0

    lax.fori_loop(0, (nblk + 2) // 3, step, 0)
    for k in range(3):
        wait_scatter(k)
    plsc.subcore_barrier()
    pltpu.sync_copy(accs.at[pl.ds(r0, _RPS)], acc_o.at[cid, pl.ds(r0, _RPS)])
    pltpu.sync_copy(dens.at[pl.ds(r0, _RPS)], den_o.at[cid, pl.ds(r0, _RPS)])

  return _edge_pass


def _attn_mats(a_src, a_dst):
    eye = jnp.eye(_H, dtype=f32)
    a_s = a_src.reshape(_H, _C)
    a_d = a_dst.reshape(_H, _C)
    As8 = (a_s[:, :, None] * eye[:, None, :]).reshape(_HC, _H)
    Ad8 = (a_d[:, :, None] * eye[:, None, :]).reshape(_HC, _H)
    return (jnp.concatenate([As8, As8], axis=1),
            jnp.concatenate([Ad8, Ad8], axis=1))


def _expand_mat():
    e8 = jnp.kron(jnp.eye(_H, dtype=f32), jnp.ones((1, _C), f32)) * 0.5
    return jnp.concatenate([e8, e8], axis=0)


def kernel(x, edge_index, batch, W1, a_src1, a_dst1, b1, W2, a_src2, a_dst2,
           b2, W3, a_src3, a_dst3, b3, fc_w, fc_b):
    src2d = edge_index[0].reshape(_E // _BB, _BB)
    dst2d = edge_index[1].reshape(_E // _BB, _BB)
    xp = jnp.pad(x, ((0, _NPAD - _N), (0, 0)))
    expand = _expand_mat()
    batch3d = jnp.pad(batch, (0, _NPAD - _N), constant_values=_G).reshape(
        _NRB, 1, _RB)

    As1, Ad1 = _attn_mats(a_src1, a_dst1)
    As2, Ad2 = _attn_mats(a_src2, a_dst2)
    As3, Ad3 = _attn_mats(a_src3, a_dst3)
    edge_pass = _make_edge_pass()

    Ws = jnp.stack([W1, W2, W3])
    Ass = jnp.stack([As1, As2, As3])
    Ads = jnp.stack([Ad1, Ad2, Ad3])
    bs = jnp.stack([jnp.zeros_like(b1), b1, b2]).reshape(3, 1, _HC)

    def body(carry, xs):
        pa, pd = carry
        Wl, Asl, Adl, bl, first = xs
        h, s, d = lax.cond(
            first,
            lambda: _dense_call(xp, Wl, Asl, Adl),
            lambda: _mid_call(pa, pd, bl, expand, Wl, Asl, Adl),
        )
        pa2, pd2 = edge_pass(src2d, dst2d, h, s, d)
        return (pa2, pd2), None

    pa0 = jnp.zeros((_NC, _NPAD, _HC), f32)
    pd0 = jnp.zeros((_NC, _NPAD, 16), f32)
    (pa, pd), _ = lax.scan(
        body, (pa0, pd0),
        (Ws, Ass, Ads, bs, jnp.array([True, False, False])))
    out = _pool_call(pa, pd, b3.reshape(1, _HC), expand, batch3d,
                     fc_w.reshape(1, _HC), fc_b.reshape(1, 1))
    return out


# speedup vs baseline: 1.1394x; 1.1394x over previous
import functools

import jax
import jax.numpy as jnp
from jax import lax
from jax.experimental import pallas as pl
from jax.experimental.pallas import tpu as pltpu
from jax.experimental.pallas import tpu_sc as plsc

_N = 10000
_E = 320000
_H = 8
_C = 16
_HC = 128
_G = 64

_NPAD = 10112
_NC = 2
_NS = 16
_NW = _NC * _NS
_EPW = 10240
_BB = 64
_NBLK = _EPW // _BB
_GB = 8
_RPS = _NPAD // _NS

_RB = 128
_NRB = _NPAD // _RB

f32 = jnp.float32
i32 = jnp.int32


def _dense_body(x_ref, w_ref, as_ref, ad_ref, h_ref, s_ref, d_ref):
    h = jnp.dot(x_ref[...], w_ref[...], preferred_element_type=f32)
    h_ref[...] = h
    s_ref[...] = jnp.dot(h, as_ref[...], preferred_element_type=f32)
    d_ref[...] = jnp.dot(h, ad_ref[...], preferred_element_type=f32)


_dense_call = pl.pallas_call(
    _dense_body,
    grid=(_NRB,),
    in_specs=[
        pl.BlockSpec((_RB, _HC), lambda i: (i, 0)),
        pl.BlockSpec((_HC, _HC), lambda i: (0, 0)),
        pl.BlockSpec((_HC, 16), lambda i: (0, 0)),
        pl.BlockSpec((_HC, 16), lambda i: (0, 0)),
    ],
    out_specs=[
        pl.BlockSpec((_RB, _HC), lambda i: (i, 0)),
        pl.BlockSpec((_RB, 16), lambda i: (i, 0)),
        pl.BlockSpec((_RB, 16), lambda i: (i, 0)),
    ],
    out_shape=[
        jax.ShapeDtypeStruct((_NPAD, _HC), f32),
        jax.ShapeDtypeStruct((_NPAD, 16), f32),
        jax.ShapeDtypeStruct((_NPAD, 16), f32),
    ],
)


def _mid_body(pa_ref, pd_ref, bias_ref, exp_ref, w_ref, as_ref, ad_ref,
              h_ref, s_ref, d_ref):
    acc = pa_ref[0] + pa_ref[1]
    den16 = pd_ref[0] + pd_ref[1]
    den = jnp.dot(den16, exp_ref[...], preferred_element_type=f32)
    x_in = acc / (den + 1e-16) + bias_ref[...]
    x_in = jnp.maximum(x_in, 0.0)
    h = jnp.dot(x_in, w_ref[...], preferred_element_type=f32)
    h_ref[...] = h
    s_ref[...] = jnp.dot(h, as_ref[...], preferred_element_type=f32)
    d_ref[...] = jnp.dot(h, ad_ref[...], preferred_element_type=f32)


_mid_call = pl.pallas_call(
    _mid_body,
    grid=(_NRB,),
    in_specs=[
        pl.BlockSpec((_NC, _RB, _HC), lambda i: (0, i, 0)),
        pl.BlockSpec((_NC, _RB, 16), lambda i: (0, i, 0)),
        pl.BlockSpec((1, _HC), lambda i: (0, 0)),
        pl.BlockSpec((16, _HC), lambda i: (0, 0)),
        pl.BlockSpec((_HC, _HC), lambda i: (0, 0)),
        pl.BlockSpec((_HC, 16), lambda i: (0, 0)),
        pl.BlockSpec((_HC, 16), lambda i: (0, 0)),
    ],
    out_specs=[
        pl.BlockSpec((_RB, _HC), lambda i: (i, 0)),
        pl.BlockSpec((_RB, 16), lambda i: (i, 0)),
        pl.BlockSpec((_RB, 16), lambda i: (i, 0)),
    ],
    out_shape=[
        jax.ShapeDtypeStruct((_NPAD, _HC), f32),
        jax.ShapeDtypeStruct((_NPAD, 16), f32),
        jax.ShapeDtypeStruct((_NPAD, 16), f32),
    ],
)


def _pool_body(pa_ref, pd_ref, bias_ref, exp_ref, batch_ref, fcw_ref, fcb_ref,
               out_ref, sums_ref, cnt_ref):
    i = pl.program_id(0)

    @pl.when(i == 0)
    def _():
        sums_ref[...] = jnp.zeros_like(sums_ref)
        cnt_ref[...] = jnp.zeros_like(cnt_ref)

    acc = pa_ref[0] + pa_ref[1]
    den16 = pd_ref[0] + pd_ref[1]
    den = jnp.dot(den16, exp_ref[...], preferred_element_type=f32)
    node = acc / (den + 1e-16) + bias_ref[...]
    b_ids = batch_ref[0]
    gids = lax.broadcasted_iota(i32, (_G, _RB), 0)
    mask = (b_ids == gids).astype(f32)
    sums_ref[...] += jnp.dot(mask, node, preferred_element_type=f32)
    cnt_ref[...] += jnp.sum(mask, axis=1, keepdims=True)

    @pl.when(i == _NRB - 1)
    def _():
        pooled = sums_ref[...] / jnp.maximum(cnt_ref[...], 1.0)
        out_ref[...] = (jnp.sum(pooled * fcw_ref[...], axis=1, keepdims=True)
                        + fcb_ref[0, 0])


_pool_call = pl.pallas_call(
    _pool_body,
    grid=(_NRB,),
    in_specs=[
        pl.BlockSpec((_NC, _RB, _HC), lambda i: (0, i, 0)),
        pl.BlockSpec((_NC, _RB, 16), lambda i: (0, i, 0)),
        pl.BlockSpec((1, _HC), lambda i: (0, 0)),
        pl.BlockSpec((16, _HC), lambda i: (0, 0)),
        pl.BlockSpec((1, 1, _RB), lambda i: (i, 0, 0)),
        pl.BlockSpec((1, _HC), lambda i: (0, 0)),
        pl.BlockSpec((1, 1), lambda i: (0, 0)),
    ],
    out_specs=pl.BlockSpec((_G, 1), lambda i: (0, 0)),
    out_shape=jax.ShapeDtypeStruct((_G, 1), f32),
    scratch_shapes=[
        pltpu.VMEM((_G, _HC), f32),
        pltpu.VMEM((_G, 1), f32),
    ],
)


@functools.lru_cache(maxsize=None)
def _make_edge_pass():
  mesh = plsc.VectorSubcoreMesh(core_axis_name="c", subcore_axis_name="s",
                                num_cores=_NC, num_subcores=_NS)

  ring_buf = [
      pltpu.VMEM((_BB,), i32),
      pltpu.VMEM((_BB,), i32),
      pltpu.VMEM((_BB, 16), f32),
      pltpu.VMEM((_BB, 16), f32),
      pltpu.VMEM((_BB, _HC), f32),
      pltpu.VMEM((_BB, 16), f32),
      pltpu.SemaphoreType.DMA,
      pltpu.SemaphoreType.DMA,
      pltpu.SemaphoreType.DMA,
  ]

  @functools.partial(
    pl.kernel,
    out_type=(jax.ShapeDtypeStruct((_NC, _NPAD, _HC), f32),
              jax.ShapeDtypeStruct((_NC, _NPAD, 16), f32)),
    mesh=mesh,
    compiler_params=pltpu.CompilerParams(use_tc_tiling_on_sc=False),
    scratch_types=[
        pltpu.VMEM((_GB, _BB), i32),
        pltpu.VMEM((_GB, _BB), i32),
        pltpu.VMEM_SHARED((_NPAD, _HC), f32),
        pltpu.VMEM_SHARED((_NPAD, 16), f32),
    ] + ring_buf * 3,
  )
  def _edge_pass(src_h, dst_h, h_h, s_h, d_h, acc_o, den_o,
                 srcg, dstg, accs, dens, *ring):
    bufs = [ring[9 * k:9 * (k + 1)] for k in range(3)]
    cid = lax.axis_index("c")
    sid = lax.axis_index("s")
    wid = sid * _NC + cid

    hr0, wb0 = bufs[0][4], bufs[0][5]
    zv = jnp.zeros((16,), f32)

    def zi(k, _):
        for j in range(_HC // 16):
            hr0[k, pl.ds(16 * j, 16)] = zv
        wb0[k, :] = zv
        return 0

    lax.fori_loop(0, _BB, zi, 0)
    r0 = sid * _RPS
    for t in range(_RPS // _BB):
        pltpu.sync_copy(hr0, accs.at[pl.ds(r0 + t * _BB, _BB)])
        pltpu.sync_copy(wb0, dens.at[pl.ds(r0 + t * _BB, _BB)])
    rem = _RPS - (_RPS // _BB) * _BB
    pltpu.sync_copy(hr0.at[pl.ds(0, rem)],
                    accs.at[pl.ds(r0 + _RPS - rem, rem)])
    pltpu.sync_copy(wb0.at[pl.ds(0, rem)],
                    dens.at[pl.ds(r0 + _RPS - rem, rem)])
    plsc.subcore_barrier()

    base_blk = wid * _NBLK
    nblk = jnp.minimum(_NBLK, jnp.maximum(0, (_E - wid * _EPW) // _BB))

    def load_group(first_blk):
        g = pl.multiple_of(base_blk + first_blk, _GB)
        pltpu.sync_copy(src_h.at[pl.ds(g, _GB)], srcg)
        pltpu.sync_copy(dst_h.at[pl.ds(g, _GB)], dstg)

    def stage(j, k):
        sv, dv = bufs[k][0], bufs[k][1]
        row = lax.rem(j, _GB)
        for m in range(_BB // 16):
            sl = pl.ds(16 * m, 16)
            sv[sl] = srcg[row, sl]
            dv[sl] = dstg[row, sl]
        sr, dr, hr = bufs[k][2], bufs[k][3], bufs[k][4]
        g_sd, g_h = bufs[k][6], bufs[k][7]
        pltpu.async_copy(s_h.at[sv], sr, g_sd)
        pltpu.async_copy(d_h.at[dv], dr, g_sd)
        pltpu.async_copy(h_h.at[sv], hr, g_h)

    def wait_gathers_sd(k):
        sv, dv, sr, dr = bufs[k][0], bufs[k][1], bufs[k][2], bufs[k][3]
        g_sd = bufs[k][6]
        pltpu.make_async_copy(s_h.at[sv], sr, g_sd).wait()
        pltpu.make_async_copy(d_h.at[dv], dr, g_sd).wait()

    def wait_gather_h(k):
        sv, hr, g_h = bufs[k][0], bufs[k][4], bufs[k][7]
        pltpu.make_async_copy(h_h.at[sv], hr, g_h).wait()

    def issue_scatter(k):
        dv, hr, wb, g_sc = bufs[k][1], bufs[k][4], bufs[k][5], bufs[k][8]
        pltpu.async_copy(hr, accs.at[dv], g_sc, add=True)
        pltpu.async_copy(wb, dens.at[dv], g_sc, add=True)

    def wait_scatter(k):
        dv, hr, wb, g_sc = bufs[k][1], bufs[k][4], bufs[k][5], bufs[k][8]
        pltpu.make_async_copy(hr, accs.at[dv], g_sc).wait()
        pltpu.make_async_copy(wb, dens.at[dv], g_sc).wait()

    def compute(k):
        sr, dr, hr, wb = bufs[k][2], bufs[k][3], bufs[k][4], bufs[k][5]

        @plsc.parallel_loop(0, _BB, unroll=4)
        def edge_w(b):
            e = sr[b, :] + dr[b, :]
            e = jnp.where(e > 0, e, f32(0.2) * e)
            wb[b, :] = jnp.exp(e)

        wait_gather_h(k)

        def edge_m(b, _):
            w = wb[b, :]
            for j in range(_H):
                hr[b, pl.ds(16 * j, 16)] = hr[b, pl.ds(16 * j, 16)] * w[j]
            return 0

        lax.fori_loop(0, _BB, edge_m, 0)

    load_group(0)
    stage(0, 0)
    stage(1, 1)

    def step(i3, _):
        for k in range(3):
            j = 3 * i3 + k

            @pl.when(j < nblk)
            def _():
                k2 = (k + 2) % 3
                wait_gathers_sd(k)
                compute(k)

                @pl.when(j + 2 < nblk)
                def _():
                    @pl.when(j >= 1)
                    def _():
                        wait_scatter(k2)

                    @pl.when(lax.rem(j + 2, _GB) == 0)
                    def _():
                        load_group(j + 2)

                    stage(j + 2, k2)

                issue_scatter(k)

        return 0

    lax.fori_loop(0, (nblk + 2) // 3, step, 0)
    for k in range(3):
        wait_scatter(k)
    plsc.subcore_barrier()
    pltpu.sync_copy(accs.at[pl.ds(r0, _RPS)], acc_o.at[cid, pl.ds(r0, _RPS)])
    pltpu.sync_copy(dens.at[pl.ds(r0, _RPS)], den_o.at[cid, pl.ds(r0, _RPS)])

  return _edge_pass


def _attn_mats(a_src, a_dst):
    eye = jnp.eye(_H, dtype=f32)
    a_s = a_src.reshape(_H, _C)
    a_d = a_dst.reshape(_H, _C)
    As8 = (a_s[:, :, None] * eye[:, None, :]).reshape(_HC, _H)
    Ad8 = (a_d[:, :, None] * eye[:, None, :]).reshape(_HC, _H)
    return (jnp.concatenate([As8, As8], axis=1),
            jnp.concatenate([Ad8, Ad8], axis=1))


def _expand_mat():
    e8 = jnp.kron(jnp.eye(_H, dtype=f32), jnp.ones((1, _C), f32)) * 0.5
    return jnp.concatenate([e8, e8], axis=0)


def kernel(x, edge_index, batch, W1, a_src1, a_dst1, b1, W2, a_src2, a_dst2,
           b2, W3, a_src3, a_dst3, b3, fc_w, fc_b):
    src2d = edge_index[0].reshape(_E // _BB, _BB)
    dst2d = edge_index[1].reshape(_E // _BB, _BB)
    xp = jnp.pad(x, ((0, _NPAD - _N), (0, 0)))
    expand = _expand_mat()
    batch3d = jnp.pad(batch, (0, _NPAD - _N), constant_values=_G).reshape(
        _NRB, 1, _RB)

    As1, Ad1 = _attn_mats(a_src1, a_dst1)
    As2, Ad2 = _attn_mats(a_src2, a_dst2)
    As3, Ad3 = _attn_mats(a_src3, a_dst3)
    edge_pass = _make_edge_pass()

    Ws = jnp.stack([W1, W2, W3])
    Ass = jnp.stack([As1, As2, As3])
    Ads = jnp.stack([Ad1, Ad2, Ad3])
    bs = jnp.stack([jnp.zeros_like(b1), b1, b2]).reshape(3, 1, _HC)

    def body(carry, xs):
        pa, pd = carry
        Wl, Asl, Adl, bl, first = xs
        h, s, d = lax.cond(
            first,
            lambda: _dense_call(xp, Wl, Asl, Adl),
            lambda: _mid_call(pa, pd, bl, expand, Wl, Asl, Adl),
        )
        pa2, pd2 = edge_pass(src2d, dst2d, h, s, d)
        return (pa2, pd2), None

    pa0 = jnp.zeros((_NC, _NPAD, _HC), f32)
    pd0 = jnp.zeros((_NC, _NPAD, 16), f32)
    (pa, pd), _ = lax.scan(
        body, (pa0, pd0),
        (Ws, Ass, Ads, bs, jnp.array([True, False, False])))
    out = _pool_call(pa, pd, b3.reshape(1, _HC), expand, batch3d,
                     fc_w.reshape(1, _HC), fc_b.reshape(1, 1))
    return out
